# baseline (speedup 1.0000x reference)
import functools

import jax
import jax.numpy as jnp
from jax import lax
from jax.experimental import pallas as pl
from jax.experimental.pallas import tpu as pltpu
from jax.experimental.pallas import tpu_sc as plsc

_NC = 2
_NS = 16
_NW = _NC * _NS
_LANES = 16


@functools.partial(jax.jit, static_argnums=(2, 3, 4))
def _sc_pos_add(x, pos_weight, B, T, D):
    t_per = T // _NW
    CH = 16
    n_ch = t_per // CH
    n_vec_row = D // _LANES
    UNROLL = 8
    n_steps = n_ch * B
    n_g = n_steps // 2

    mesh = plsc.VectorSubcoreMesh(core_axis_name="c", subcore_axis_name="s")

    @functools.partial(
        pl.kernel,
        mesh=mesh,
        out_type=jax.ShapeDtypeStruct((B, T, D), jnp.float32),
        scratch_types=[
            pltpu.VMEM((CH, D), jnp.float32),
            pltpu.VMEM((CH, D), jnp.float32),
            pltpu.VMEM((CH, D), jnp.float32),
            pltpu.VMEM((CH, D), jnp.float32),
            pltpu.VMEM((CH, D), jnp.float32),
            pltpu.SemaphoreType.DMA,
            pltpu.SemaphoreType.DMA,
            pltpu.SemaphoreType.DMA,
            pltpu.SemaphoreType.DMA,
        ],
    )
    def k(x_hbm, pos_hbm, out_hbm, pos_v, x0, x1, o0, o1, sx0, sx1, so0, so1):
        w = lax.axis_index("s") * _NC + lax.axis_index("c")
        t0 = w * t_per
        xbufs, obufs = (x0, x1), (o0, o1)
        sxs, sos = (sx0, sx1), (so0, so1)

        def coords(s):
            c = s // B
            b = s - c * B
            return b, t0 + c * CH

        b0, r0 = coords(0)
        b1, r1 = coords(1)
        pltpu.async_copy(x_hbm.at[b0, pl.ds(r0, CH)], x0, sx0)
        pltpu.async_copy(x_hbm.at[b1, pl.ds(r1, CH)], x1, sx1)

        def g_body(g, _):
            for u in (0, 1):
                s = g * 2 + u
                b, row0 = coords(s)
                xv, ov, sx, so = xbufs[u], obufs[u], sxs[u], sos[u]

                @pl.when(b == 0)
                def _():
                    pltpu.sync_copy(pos_hbm.at[pl.ds(row0, CH)], pos_v)

                pltpu.make_async_copy(x_hbm.at[0, pl.ds(0, CH)], xv, sx).wait()

                @pl.when(g > 0)
                def _():
                    pltpu.make_async_copy(
                        ov, out_hbm.at[0, pl.ds(0, CH)], so
                    ).wait()

                def row_body(r, _):
                    def add_body(i, _):
                        base = i * (_LANES * UNROLL)
                        for uu in range(UNROLL):
                            off = base + uu * _LANES
                            ov[r, pl.ds(off, _LANES)] = (
                                xv[r, pl.ds(off, _LANES)]
                                + pos_v[r, pl.ds(off, _LANES)]
                            )
                        return 0

                    lax.fori_loop(
                        0, n_vec_row // UNROLL, add_body, 0, unroll=False
                    )
                    return 0

                lax.fori_loop(0, CH, row_body, 0, unroll=False)

                pltpu.async_copy(ov, out_hbm.at[b, pl.ds(row0, CH)], so)

                @pl.when(g < n_g - 1)
                def _():
                    b2, row2 = coords(s + 2)
                    pltpu.async_copy(x_hbm.at[b2, pl.ds(row2, CH)], xv, sx)
            return 0

        lax.fori_loop(0, n_g, g_body, 0, unroll=False)
        pltpu.make_async_copy(o0, out_hbm.at[0, pl.ds(0, CH)], so0).wait()
        pltpu.make_async_copy(o1, out_hbm.at[0, pl.ds(0, CH)], so1).wait()

    return k(x, pos_weight)


def kernel(x, pos_weight):
    B, T, D = x.shape
    return _sc_pos_add(x, pos_weight, B, T, D)


# speedup vs baseline: 1.0016x; 1.0016x over previous
import functools

import jax
import jax.numpy as jnp
from jax import lax
from jax.experimental import pallas as pl
from jax.experimental.pallas import tpu as pltpu
from jax.experimental.pallas import tpu_sc as plsc

_NC = 2
_NS = 16
_NW = _NC * _NS
_LANES = 16


@functools.partial(jax.jit, static_argnums=(2, 3, 4))
def _sc_pos_add(x, pos_weight, B, T, D):
    t_per = T // _NW
    CH = 16
    n_ch = t_per // CH
    n_vec_row = D // _LANES
    UNROLL = 8
    n_steps = n_ch * B
    n_g = n_steps // 2

    mesh = plsc.VectorSubcoreMesh(core_axis_name="c", subcore_axis_name="s")

    @functools.partial(
        pl.kernel,
        mesh=mesh,
        compiler_params=pltpu.CompilerParams(use_tc_tiling_on_sc=True),
        out_type=jax.ShapeDtypeStruct((B, T, D), jnp.float32),
        scratch_types=[
            pltpu.VMEM((CH, D), jnp.float32),
            pltpu.VMEM((CH, D), jnp.float32),
            pltpu.VMEM((CH, D), jnp.float32),
            pltpu.VMEM((CH, D), jnp.float32),
            pltpu.VMEM((CH, D), jnp.float32),
            pltpu.SemaphoreType.DMA,
            pltpu.SemaphoreType.DMA,
            pltpu.SemaphoreType.DMA,
            pltpu.SemaphoreType.DMA,
        ],
    )
    def k(x_hbm, pos_hbm, out_hbm, pos_v, x0, x1, o0, o1, sx0, sx1, so0, so1):
        w = lax.axis_index("s") * _NC + lax.axis_index("c")
        t0 = w * t_per
        xbufs, obufs = (x0, x1), (o0, o1)
        sxs, sos = (sx0, sx1), (so0, so1)

        def coords(s):
            c = s // B
            b = s - c * B
            return b, t0 + c * CH

        b0, r0 = coords(0)
        b1, r1 = coords(1)
        pltpu.async_copy(x_hbm.at[b0, pl.ds(r0, CH)], x0, sx0)
        pltpu.async_copy(x_hbm.at[b1, pl.ds(r1, CH)], x1, sx1)

        def g_body(g, _):
            for u in (0, 1):
                s = g * 2 + u
                b, row0 = coords(s)
                xv, ov, sx, so = xbufs[u], obufs[u], sxs[u], sos[u]

                @pl.when(b == 0)
                def _():
                    pltpu.sync_copy(pos_hbm.at[pl.ds(row0, CH)], pos_v)

                pltpu.make_async_copy(x_hbm.at[0, pl.ds(0, CH)], xv, sx).wait()

                @pl.when(g > 0)
                def _():
                    pltpu.make_async_copy(
                        ov, out_hbm.at[0, pl.ds(0, CH)], so
                    ).wait()

                def row_body(r, _):
                    def add_body(i, _):
                        base = i * (_LANES * UNROLL)
                        for uu in range(UNROLL):
                            off = base + uu * _LANES
                            ov[r, pl.ds(off, _LANES)] = (
                                xv[r, pl.ds(off, _LANES)]
                                + pos_v[r, pl.ds(off, _LANES)]
                            )
                        return 0

                    lax.fori_loop(
                        0, n_vec_row // UNROLL, add_body, 0, unroll=False
                    )
                    return 0

                lax.fori_loop(0, CH, row_body, 0, unroll=False)

                pltpu.async_copy(ov, out_hbm.at[b, pl.ds(row0, CH)], so)

                @pl.when(g < n_g - 1)
                def _():
                    b2, row2 = coords(s + 2)
                    pltpu.async_copy(x_hbm.at[b2, pl.ds(row2, CH)], xv, sx)
            return 0

        lax.fori_loop(0, n_g, g_body, 0, unroll=False)
        pltpu.make_async_copy(o0, out_hbm.at[0, pl.ds(0, CH)], so0).wait()
        pltpu.make_async_copy(o1, out_hbm.at[0, pl.ds(0, CH)], so1).wait()

    return k(x, pos_weight)


def kernel(x, pos_weight):
    B, T, D = x.shape
    return _sc_pos_add(x, pos_weight, B, T, D)


# speedup vs baseline: 1.7638x; 1.7610x over previous
import functools

import jax
import jax.numpy as jnp
from jax import lax
from jax.experimental import pallas as pl
from jax.experimental.pallas import tpu as pltpu
from jax.experimental.pallas import tpu_sc as plsc

_NC = 2
_NS = 16
_NW = _NC * _NS
_LANES = 16


@functools.partial(jax.jit, static_argnums=(2, 3, 4))
def _sc_pos_add(x, pos_weight, B, T, D):
    t_per = T // _NW
    CH = 16
    n_ch = t_per // CH
    n_vec_row = D // _LANES
    UNROLL = 8
    n_steps = n_ch * B
    n_g = n_steps // 2

    mesh = plsc.VectorSubcoreMesh(core_axis_name="c", subcore_axis_name="s")

    @functools.partial(
        pl.kernel,
        mesh=mesh,
        out_type=jax.ShapeDtypeStruct((B, T, D), jnp.float32),
        scratch_types=[
            pltpu.VMEM((CH, D), jnp.float32),
            pltpu.VMEM((CH, D), jnp.float32),
            pltpu.VMEM((CH, D), jnp.float32),
            pltpu.VMEM((CH, D), jnp.float32),
            pltpu.VMEM((CH, D), jnp.float32),
            pltpu.SemaphoreType.DMA,
            pltpu.SemaphoreType.DMA,
            pltpu.SemaphoreType.DMA,
            pltpu.SemaphoreType.DMA,
        ],
    )
    def k(x_hbm, pos_hbm, out_hbm, pos_v, x0, x1, o0, o1, sx0, sx1, so0, so1):
        w = lax.axis_index("s") * _NC + lax.axis_index("c")
        t0 = w * t_per
        xbufs, obufs = (x0, x1), (o0, o1)
        sxs, sos = (sx0, sx1), (so0, so1)

        def coords(s):
            c = s // B
            b = s - c * B
            return b, t0 + c * CH

        b0, r0 = coords(0)
        b1, r1 = coords(1)
        pltpu.async_copy(x_hbm.at[b0, pl.ds(r0, CH)], x0, sx0)
        pltpu.async_copy(x_hbm.at[b1, pl.ds(r1, CH)], x1, sx1)

        def g_body(g, _):
            for u in (0, 1):
                s = g * 2 + u
                b, row0 = coords(s)
                xv, ov, sx, so = xbufs[u], obufs[u], sxs[u], sos[u]

                @pl.when(b == 0)
                def _():
                    pltpu.sync_copy(pos_hbm.at[pl.ds(row0, CH)], pos_v)

                pltpu.make_async_copy(x_hbm.at[0, pl.ds(0, CH)], xv, sx).wait()

                @pl.when(g > 0)
                def _():
                    pltpu.make_async_copy(
                        ov, out_hbm.at[0, pl.ds(0, CH)], so
                    ).wait()

                def add_body(j, _):
                    base = j * (_LANES * UNROLL)
                    for r in range(CH):
                        for uu in range(UNROLL):
                            off = base + uu * _LANES
                            ov[r, pl.ds(off, _LANES)] = (
                                xv[r, pl.ds(off, _LANES)]
                                + pos_v[r, pl.ds(off, _LANES)]
                            )
                    return 0

                lax.fori_loop(
                    0, n_vec_row // UNROLL, add_body, 0, unroll=False
                )

                pltpu.async_copy(ov, out_hbm.at[b, pl.ds(row0, CH)], so)

                @pl.when(g < n_g - 1)
                def _():
                    b2, row2 = coords(s + 2)
                    pltpu.async_copy(x_hbm.at[b2, pl.ds(row2, CH)], xv, sx)
            return 0

        lax.fori_loop(0, n_g, g_body, 0, unroll=False)
        pltpu.make_async_copy(o0, out_hbm.at[0, pl.ds(0, CH)], so0).wait()
        pltpu.make_async_copy(o1, out_hbm.at[0, pl.ds(0, CH)], so1).wait()

    return k(x, pos_weight)


def kernel(x, pos_weight):
    B, T, D = x.shape
    return _sc_pos_add(x, pos_weight, B, T, D)
